# Initial kernel scaffold; baseline (speedup 1.0000x reference)
import jax
import jax.numpy as jnp
from jax.experimental import pallas as pl


def kernel(feature_in, embed_in, tree):
    raise NotImplementedError("write your pallas kernel here")


# speedup vs baseline: 37.2833x; 37.2833x over previous
import numpy as np
import jax
import jax.numpy as jnp
from jax.experimental import pallas as pl
from jax.experimental.pallas import tpu as pltpu

_CH = 256


def _chunks(m):
    o = 0
    while o < m:
        l = min(_CH, m - o)
        yield o, l
        o += l


def _tree_dp_kernel(feat_ref, emb_ref, out_ref, nrm_ref, ewl_ref, ewr_ref):
    n, c = feat_ref.shape
    K = int(np.log2(n))

    for o, l in _chunks(n // 2):
        r = slice(n // 2 + o, n // 2 + o + l)
        out_ref[r, :] = feat_ref[r, :]
        nrm_ref[r, :] = jnp.ones((l, c), jnp.float32)

    pr = n // 2 - 1
    dl = emb_ref[n - 1:n, :] - emb_ref[pr:pr + 1, :]
    wl = jnp.broadcast_to(jnp.exp(-jnp.sum(dl * dl, axis=1, keepdims=True)), (1, c))
    ewl_ref[pr:pr + 1, :] = wl
    out_ref[pr:pr + 1, :] = feat_ref[pr:pr + 1, :] + wl * out_ref[n - 1:n, :]
    nrm_ref[pr:pr + 1, :] = 1.0 + wl * nrm_ref[n - 1:n, :]

    for d in range(K - 1, 0, -1):
        s = 2**d - 1
        sp, m2 = 2 ** (d - 1) - 1, 2 ** (d - 1)
        for o, l in _chunks(m2):
            rp = slice(sp + o, sp + o + l)
            rl = slice(s + 2 * o, s + 2 * o + 2 * l, 2)
            rr = slice(s + 2 * o + 1, s + 2 * o + 2 * l, 2)
            ep = emb_ref[rp, :]
            dl = emb_ref[rl, :] - ep
            dr = emb_ref[rr, :] - ep
            wl = jnp.broadcast_to(
                jnp.exp(-jnp.sum(dl * dl, axis=1, keepdims=True)), (l, c))
            wr = jnp.broadcast_to(
                jnp.exp(-jnp.sum(dr * dr, axis=1, keepdims=True)), (l, c))
            ewl_ref[rp, :] = wl
            ewr_ref[rp, :] = wr
            out_ref[rp, :] = (feat_ref[rp, :] + wl * out_ref[rl, :]
                              + wr * out_ref[rr, :])
            nrm_ref[rp, :] = 1.0 + wl * nrm_ref[rl, :] + wr * nrm_ref[rr, :]

    for d in range(1, K):
        s = 2**d - 1
        sp, m2 = 2 ** (d - 1) - 1, 2 ** (d - 1)
        for o, l in _chunks(m2):
            rp = slice(sp + o, sp + o + l)
            rl = slice(s + 2 * o, s + 2 * o + 2 * l, 2)
            rr = slice(s + 2 * o + 1, s + 2 * o + 2 * l, 2)
            wl = ewl_ref[rp, :]
            wr = ewr_ref[rp, :]
            p = out_ref[rp, :]
            pn = nrm_ref[rp, :]
            al = out_ref[rl, :]
            ar = out_ref[rr, :]
            out_ref[rl, :] = al + wl * (p - wl * al)
            out_ref[rr, :] = ar + wr * (p - wr * ar)
            nl = nrm_ref[rl, :]
            nr = nrm_ref[rr, :]
            nrm_ref[rl, :] = nl + wl * (pn - wl * nl)
            nrm_ref[rr, :] = nr + wr * (pn - wr * nr)
    wl = ewl_ref[pr:pr + 1, :]
    a = out_ref[n - 1:n, :]
    out_ref[n - 1:n, :] = a + wl * (out_ref[pr:pr + 1, :] - wl * a)
    nn = nrm_ref[n - 1:n, :]
    nrm_ref[n - 1:n, :] = nn + wl * (nrm_ref[pr:pr + 1, :] - wl * nn)

    for o, l in _chunks(n):
        r = slice(o, o + l)
        out_ref[r, :] = out_ref[r, :] / nrm_ref[r, :]


def _run(feat_t, emb_t):
    n, c = feat_t.shape
    return pl.pallas_call(
        _tree_dp_kernel,
        out_shape=jax.ShapeDtypeStruct((n, c), jnp.float32),
        scratch_shapes=[
            pltpu.VMEM((n, c), jnp.float32),
            pltpu.VMEM((n // 2, c), jnp.float32),
            pltpu.VMEM((n // 2, c), jnp.float32),
        ],
    )(feat_t, emb_t)


def kernel(feature_in, embed_in, tree):
    b, c, h, w = feature_in.shape
    n = h * w
    ce = embed_in.shape[1]
    feat_t = jnp.transpose(feature_in.reshape(b, c, n), (0, 2, 1))
    emb_t = jnp.transpose(embed_in.reshape(b, ce, n), (0, 2, 1))
    out_t = jnp.stack([_run(feat_t[i], emb_t[i]) for i in range(b)])
    return jnp.transpose(out_t, (0, 2, 1)).reshape(b, c, h, w)
